# Initial kernel scaffold; baseline (speedup 1.0000x reference)
import jax
import jax.numpy as jnp
from jax.experimental import pallas as pl


def kernel(xyz1, xyz2, points1, points2, params):
    raise NotImplementedError("write your pallas kernel here")


# speedup vs baseline: 1.0002x; 1.0002x over previous
import math

import jax
import jax.numpy as jnp
from jax.experimental import pallas as pl

C_IN = 64
NSAMPLE = 16
VOXEL = 0.25


def _dense(x, l):
    return x @ l["W"].T + l["b"]


def _bn(x, p, eps=1e-05):
    m = jnp.mean(x, axis=(0, 1), keepdims=True)
    v = jnp.var(x, axis=(0, 1), keepdims=True)
    return (x - m) / jnp.sqrt(v + eps) * p["g"] + p["b"]


def _leaky(x):
    return jnp.where(x > 0, x, 0.1 * x)


def _knn(q, base, k):
    d = jnp.sum(q * q, -1)[:, :, None] - 2.0 * jnp.einsum("bnc,bmc->bnm", q, base) + jnp.sum(base * base, -1)[:, None, :]
    _, idx = jax.lax.top_k(-d, k)
    return idx


def _gather(pts, idx):
    return jax.vmap(lambda p, i: p[i])(pts, idx)


def _posenc(pe_raw, pec):
    scale = 2.0 * math.pi
    e = pe_raw / (1.0 + 1e-06) * scale
    i = jnp.arange(8, dtype=jnp.float32)
    dim_t = 10000.0 ** (2.0 * jnp.floor(i / 2.0) / 8.0)
    outs = []
    for c in range(3):
        p = e[:, c:c + 1] / dim_t
        p = jnp.stack([jnp.sin(p[:, 0::2]), jnp.cos(p[:, 1::2])], axis=2).reshape(p.shape[0], -1)
        outs.append(p)
    pos = jnp.concatenate(outs, axis=1)
    return _dense(pos, pec)


def _intra_patch(pos_diff, wp):
    Bq, Nq, S, _ = pos_diff.shape
    pd = pos_diff.reshape(-1, S, 3)
    r = VOXEL
    dis_voxel = jnp.round(pd / r)
    h = _dense(pd, wp["ie1"])
    h = jax.nn.relu(_bn(h, wp["ie_bn"]))
    h = _dense(h, wp["ie2"])
    pe_raw = ((pd - dis_voxel * r) / r).reshape(-1, 3)
    h = h + _posenc(pe_raw, wp["pec"]).reshape(pd.shape[0], S, -1)
    h = jax.nn.relu(_bn(_dense(h, wp["pm1"]), wp["pm1_bn"], 0.001))
    h = jax.nn.relu(_bn(_dense(h, wp["pm2"]), wp["pm2_bn"], 0.001))
    h = _bn(_dense(h, wp["pm3"]), wp["pm3_bn"], 0.001)
    h = _bn(_dense(h, wp["pm4"]), wp["pm4_bn"], 0.001)
    attn = jax.nn.softmax(h[:, :, 0], axis=-1)
    return attn.reshape(Bq, Nq, S)


def _run_mlp(x, layers):
    for l in layers:
        x = _leaky(_dense(x, l))
    return x


def kernel(xyz1, xyz2, points1, points2, params):
    x1 = xyz1.transpose(0, 2, 1)
    x2 = xyz2.transpose(0, 2, 1)
    f1 = points1.transpose(0, 2, 1)
    f2 = points2.transpose(0, 2, 1)
    K = NSAMPLE
    idx12 = _knn(x1, x2, K)
    nx2 = _gather(x2, idx12)
    dir12 = nx2 - x1[:, :, None, :]
    gf2 = _gather(f2, idx12)
    gf1 = jnp.broadcast_to(f1[:, :, None, :], gf2.shape)
    c1 = _run_mlp(jnp.concatenate([gf1, gf2], -1), params["mlp1"])
    attn1 = _intra_patch(dir12, params["wn2"])
    cost1 = jnp.sum(attn1[..., None] * c1, axis=2)

    idx21 = _knn(x2, x1, K)
    nx1 = _gather(x1, idx21)
    dir21 = nx1 - x2[:, :, None, :]
    gf1b = _gather(f1, idx21)
    gf2b = jnp.broadcast_to(f2[:, :, None, :], gf1b.shape)
    c2 = _run_mlp(jnp.concatenate([gf2b, gf1b], -1), params["mlp2"])
    attn2 = _intra_patch(dir21, params["wn2"])
    cost2 = jnp.sum(attn2[..., None] * c2, axis=2)

    gc2 = _gather(cost2, idx12)
    c3 = _run_mlp(jnp.concatenate([gc2, dir12], -1), params["mlp3"])
    cost21 = jnp.sum(attn1[..., None] * c3, axis=2)

    c4 = _run_mlp(jnp.concatenate([cost1, cost21], -1), params["mlp4"])
    flow = jnp.sum(attn1[..., None] * dir12, axis=2)
    return c4.transpose(0, 2, 1), flow.transpose(0, 2, 1)


# speedup vs baseline: 2.2820x; 2.2815x over previous
import math

import jax
import jax.numpy as jnp
from jax.experimental import pallas as pl
from jax.experimental.pallas import tpu as pltpu

C_IN = 64
NSAMPLE = 16
VOXEL = 0.25

_QB = 256


def _knn_body(qmat_ref, bmat_ref, out_ref):
    q = qmat_ref[0]
    bm = bmat_ref[0]
    d = jnp.dot(q, bm, preferred_element_type=jnp.float32)
    iota = jax.lax.broadcasted_iota(jnp.int32, d.shape, 1)
    big = jnp.int32(2 ** 30)
    for r in range(NSAMPLE):
        m = jnp.min(d, axis=1, keepdims=True)
        sel = jnp.min(jnp.where(d <= m, iota, big), axis=1, keepdims=True)
        out_ref[0, :, r:r + 1] = sel
        d = jnp.where(iota == sel, jnp.float32(jnp.inf), d)


def _knn_pallas(x1, x2):
    B, N, _ = x1.shape
    q_all = jnp.concatenate([x1, x2], axis=0)
    b_all = jnp.concatenate([x2, x1], axis=0)
    ones = jnp.ones((2 * B, N, 1), jnp.float32)
    zeros = jnp.zeros((2 * B, N, 4), jnp.float32)
    qmat = jnp.concatenate([q_all, ones, zeros], axis=-1)
    bb = jnp.sum(b_all * b_all, axis=-1, keepdims=True)
    bmat_rows = jnp.concatenate([-2.0 * b_all, bb, zeros], axis=-1)
    bmat = bmat_rows.transpose(0, 2, 1)

    grid = (2 * B, N // _QB)
    out = pl.pallas_call(
        _knn_body,
        grid=grid,
        in_specs=[
            pl.BlockSpec((1, _QB, 8), lambda g, i: (g, i, 0)),
            pl.BlockSpec((1, 8, N), lambda g, i: (g, 0, 0)),
        ],
        out_specs=pl.BlockSpec((1, _QB, NSAMPLE), lambda g, i: (g, i, 0)),
        out_shape=jax.ShapeDtypeStruct((2 * B, N, NSAMPLE), jnp.int32),
    )(qmat, bmat)
    return out[:B], out[B:]


def _dense(x, l):
    return x @ l["W"].T + l["b"]


def _bn(x, p, eps=1e-05):
    m = jnp.mean(x, axis=(0, 1), keepdims=True)
    v = jnp.var(x, axis=(0, 1), keepdims=True)
    return (x - m) / jnp.sqrt(v + eps) * p["g"] + p["b"]


def _leaky(x):
    return jnp.where(x > 0, x, 0.1 * x)


def _gather(pts, idx):
    return jax.vmap(lambda p, i: p[i])(pts, idx)


def _posenc(pe_raw, pec):
    scale = 2.0 * math.pi
    e = pe_raw / (1.0 + 1e-06) * scale
    i = jnp.arange(8, dtype=jnp.float32)
    dim_t = 10000.0 ** (2.0 * jnp.floor(i / 2.0) / 8.0)
    outs = []
    for c in range(3):
        p = e[:, c:c + 1] / dim_t
        p = jnp.stack([jnp.sin(p[:, 0::2]), jnp.cos(p[:, 1::2])], axis=2).reshape(p.shape[0], -1)
        outs.append(p)
    pos = jnp.concatenate(outs, axis=1)
    return _dense(pos, pec)


def _intra_patch(pos_diff, wp):
    Bq, Nq, S, _ = pos_diff.shape
    pd = pos_diff.reshape(-1, S, 3)
    r = VOXEL
    dis_voxel = jnp.round(pd / r)
    h = _dense(pd, wp["ie1"])
    h = jax.nn.relu(_bn(h, wp["ie_bn"]))
    h = _dense(h, wp["ie2"])
    pe_raw = ((pd - dis_voxel * r) / r).reshape(-1, 3)
    h = h + _posenc(pe_raw, wp["pec"]).reshape(pd.shape[0], S, -1)
    h = jax.nn.relu(_bn(_dense(h, wp["pm1"]), wp["pm1_bn"], 0.001))
    h = jax.nn.relu(_bn(_dense(h, wp["pm2"]), wp["pm2_bn"], 0.001))
    h = _bn(_dense(h, wp["pm3"]), wp["pm3_bn"], 0.001)
    h = _bn(_dense(h, wp["pm4"]), wp["pm4_bn"], 0.001)
    attn = jax.nn.softmax(h[:, :, 0], axis=-1)
    return attn.reshape(Bq, Nq, S)


def _run_mlp(x, layers):
    for l in layers:
        x = _leaky(_dense(x, l))
    return x


def kernel(xyz1, xyz2, points1, points2, params):
    x1 = xyz1.transpose(0, 2, 1)
    x2 = xyz2.transpose(0, 2, 1)
    f1 = points1.transpose(0, 2, 1)
    f2 = points2.transpose(0, 2, 1)
    idx12, idx21 = _knn_pallas(x1, x2)

    nx2 = _gather(x2, idx12)
    dir12 = nx2 - x1[:, :, None, :]
    gf2 = _gather(f2, idx12)
    gf1 = jnp.broadcast_to(f1[:, :, None, :], gf2.shape)
    c1 = _run_mlp(jnp.concatenate([gf1, gf2], -1), params["mlp1"])
    attn1 = _intra_patch(dir12, params["wn2"])
    cost1 = jnp.sum(attn1[..., None] * c1, axis=2)

    nx1 = _gather(x1, idx21)
    dir21 = nx1 - x2[:, :, None, :]
    gf1b = _gather(f1, idx21)
    gf2b = jnp.broadcast_to(f2[:, :, None, :], gf1b.shape)
    c2 = _run_mlp(jnp.concatenate([gf2b, gf1b], -1), params["mlp2"])
    attn2 = _intra_patch(dir21, params["wn2"])
    cost2 = jnp.sum(attn2[..., None] * c2, axis=2)

    gc2 = _gather(cost2, idx12)
    c3 = _run_mlp(jnp.concatenate([gc2, dir12], -1), params["mlp3"])
    cost21 = jnp.sum(attn1[..., None] * c3, axis=2)

    c4 = _run_mlp(jnp.concatenate([cost1, cost21], -1), params["mlp4"])
    flow = jnp.sum(attn1[..., None] * dir12, axis=2)
    return c4.transpose(0, 2, 1), flow.transpose(0, 2, 1)
